# Initial kernel scaffold; baseline (speedup 1.0000x reference)
import jax
import jax.numpy as jnp
from jax.experimental import pallas as pl


def kernel(x, centroids, centroid_norm):
    raise NotImplementedError("write your pallas kernel here")


# speedup vs baseline: 1.2463x; 1.2463x over previous
import jax
import jax.numpy as jnp
from jax.experimental import pallas as pl

_NUM_WINDOWS = 4


def _kmeans_body(x_ref, c_ref, cn_ref, out_ref):
    xb = x_ref[...]
    k = c_ref.shape[1]
    kw = k // _NUM_WINDOWS
    mm = jnp.dot(xb, c_ref[...], preferred_element_type=jnp.float32)
    dist = jnp.sum(xb * xb, axis=1, keepdims=True) - 2.0 * mm + cn_ref[...]
    acc_v = jnp.full((xb.shape[0],), jnp.inf, dtype=jnp.float32)
    acc_i = jnp.zeros((xb.shape[0],), dtype=jnp.int32)
    for w in range(_NUM_WINDOWS):
        chunk = dist[:, w * kw:(w + 1) * kw]
        wmin = jnp.min(chunk, axis=1)
        widx = jnp.argmin(chunk, axis=1).astype(jnp.int32) + w * kw
        take = wmin < acc_v
        acc_i = jnp.where(take, widx, acc_i)
        acc_v = jnp.where(take, wmin, acc_v).astype(jnp.bfloat16).astype(jnp.float32)
    out_ref[...] = acc_i


def kernel(x, centroids, centroid_norm):
    n, d = x.shape
    k = centroids.shape[1]
    bn = 256
    return pl.pallas_call(
        _kmeans_body,
        grid=(n // bn,),
        in_specs=[
            pl.BlockSpec((bn, d), lambda i: (i, 0)),
            pl.BlockSpec((d, k), lambda i: (0, 0)),
            pl.BlockSpec((1, k), lambda i: (0, 0)),
        ],
        out_specs=pl.BlockSpec((bn,), lambda i: (i,)),
        out_shape=jax.ShapeDtypeStruct((n,), jnp.int32),
    )(x, centroids, centroid_norm)


# speedup vs baseline: 1.5803x; 1.2680x over previous
import jax
import jax.numpy as jnp
from jax.experimental import pallas as pl

_NUM_WINDOWS = 4


def _kmeans_body(x_ref, c_ref, cn_ref, out_ref):
    xb = x_ref[...]
    k = c_ref.shape[1]
    kw = k // _NUM_WINDOWS
    bn = xb.shape[0]
    lanes = 128
    tiles = kw // lanes
    xsq = jnp.sum(xb * xb, axis=1, keepdims=True)
    acc_v = jnp.full((bn,), jnp.inf, dtype=jnp.float32)
    acc_i = jnp.zeros((bn,), dtype=jnp.int32)
    lane_iota = jax.lax.broadcasted_iota(jnp.int32, (bn, lanes), 1)
    for w in range(_NUM_WINDOWS):
        mm = jnp.dot(xb, c_ref[:, w * kw:(w + 1) * kw],
                     preferred_element_type=jnp.float32)
        dist = xsq + mm + cn_ref[:, w * kw:(w + 1) * kw]
        run_v = jnp.full((bn, lanes), jnp.inf, dtype=jnp.float32)
        run_t = jnp.zeros((bn, lanes), dtype=jnp.int32)
        for j in range(tiles):
            dj = dist[:, j * lanes:(j + 1) * lanes]
            pred = dj < run_v
            run_v = jnp.where(pred, dj, run_v)
            run_t = jnp.where(pred, j, run_t)
        wmin = jnp.min(run_v, axis=1)
        packed = run_t * lanes + lane_iota
        widx = jnp.min(jnp.where(run_v == wmin[:, None], packed, kw),
                       axis=1) + w * kw
        take = wmin < acc_v
        acc_i = jnp.where(take, widx, acc_i)
        acc_v = jnp.where(take, wmin, acc_v).astype(jnp.bfloat16).astype(jnp.float32)
    out_ref[...] = acc_i


def kernel(x, centroids, centroid_norm):
    n, d = x.shape
    k = centroids.shape[1]
    bn = 512
    cneg = centroids * (-2.0)
    return pl.pallas_call(
        _kmeans_body,
        grid=(n // bn,),
        in_specs=[
            pl.BlockSpec((bn, d), lambda i: (i, 0)),
            pl.BlockSpec((d, k), lambda i: (0, 0)),
            pl.BlockSpec((1, k), lambda i: (0, 0)),
        ],
        out_specs=pl.BlockSpec((bn,), lambda i: (i,)),
        out_shape=jax.ShapeDtypeStruct((n,), jnp.int32),
    )(x, cneg, centroid_norm)
